# Initial kernel scaffold; baseline (speedup 1.0000x reference)
import jax
import jax.numpy as jnp
from jax.experimental import pallas as pl


def kernel(x, edge_index_0, edge_index_1, edge_index_2, edge_index_3, cluster_0, cluster_1, cluster_2, params):
    raise NotImplementedError("write your pallas kernel here")


# speedup vs baseline: 2.0347x; 2.0347x over previous
import functools

import jax
import jax.numpy as jnp
from jax import lax
from jax.experimental import pallas as pl
from jax.experimental.pallas import tpu as pltpu
from jax.experimental.pallas import tpu_sc as plsc

jax.config.update("jax_default_matmul_precision", "float32")

_ENC_CH = [32, 32, 64, 128]
_DEC_CH = [128, 64, 32, 32]
_NS = [10000, 2500, 625, 156]
_ES = [320000, 80000, 20000, 5000]

_NW = 32
_CHUNK = 128
_GRAIN = _NW * 8 * _CHUNK


def _rup(x, m):
    return (x + m - 1) // m * m


@functools.lru_cache(None)
def _seg_sum_kernel(n_src, m, n_dst, e_pad):
    C = e_pad // _CHUNK // _NW
    n_acc = _rup(n_dst + 1, 128)
    R = n_acc // 16
    mesh = plsc.VectorSubcoreMesh(core_axis_name="c", subcore_axis_name="s")

    def body(values, src2d, dst2d, zeros, out, src_v, dst_v, rows_v, acc_sh, sem):
        c = lax.axis_index("c")
        s = lax.axis_index("s")
        wid = c * 16 + s
        pltpu.sync_copy(zeros.at[pl.ds(s * R, R)], acc_sh.at[pl.ds(s * R, R)])
        pltpu.sync_copy(src2d.at[pl.ds(wid * C, C)], src_v)
        pltpu.sync_copy(dst2d.at[pl.ds(wid * C, C)], dst_v)
        plsc.subcore_barrier()

        def step(j, carry):
            pltpu.async_copy(values.at[src_v.at[j]], rows_v, sem).wait()
            pltpu.sync_copy(rows_v, acc_sh.at[dst_v.at[j]], add=True)
            return carry

        lax.fori_loop(0, C, step, 0)
        plsc.subcore_barrier()
        pltpu.sync_copy(acc_sh.at[pl.ds(s * R, R)], out.at[c, pl.ds(s * R, R)])

    return pl.kernel(
        body,
        out_type=jax.ShapeDtypeStruct((2, n_acc, m), jnp.float32),
        mesh=mesh,
        compiler_params=pltpu.CompilerParams(use_tc_tiling_on_sc=False),
        scratch_types=[
            pltpu.VMEM((C, _CHUNK), jnp.int32),
            pltpu.VMEM((C, _CHUNK), jnp.int32),
            pltpu.VMEM((_CHUNK, m), jnp.float32),
            pltpu.VMEM_SHARED((n_acc, m), jnp.float32),
            pltpu.SemaphoreType.DMA,
        ],
    )


def _seg_sum(values, src2d, dst2d, n_dst, e_pad):
    n_src, m = values.shape
    k = _seg_sum_kernel(n_src, m, n_dst, e_pad)
    n_acc = _rup(n_dst + 1, 128)
    zeros = jnp.zeros((n_acc, m), jnp.float32)
    out = k(values, src2d, dst2d, zeros)
    return out[0, :n_dst] + out[1, :n_dst]


@functools.lru_cache(None)
def _hist_kernel(n_rows, n_acc):
    C = n_rows // _NW
    R = n_acc // 16
    mesh = plsc.VectorSubcoreMesh(core_axis_name="c", subcore_axis_name="s")

    def body(dst2d, zeros, out, dst_v, ones_v, acc_sh, sem):
        c = lax.axis_index("c")
        s = lax.axis_index("s")
        wid = c * 16 + s
        pltpu.sync_copy(zeros.at[pl.ds(s * R, R)], acc_sh.at[pl.ds(s * R, R)])
        pltpu.sync_copy(dst2d.at[pl.ds(wid * C, C)], dst_v)
        for i in range(_CHUNK // 16):
            ones_v[pl.ds(i * 16, 16)] = jnp.ones((16,), jnp.float32)
        plsc.subcore_barrier()

        def step(j, carry):
            pltpu.sync_copy(ones_v, acc_sh.at[dst_v.at[j]], add=True)
            return carry

        lax.fori_loop(0, C, step, 0)
        plsc.subcore_barrier()
        pltpu.sync_copy(acc_sh.at[pl.ds(s * R, R)],
                        out.at[pl.ds(c * n_acc + s * R, R)])

    return pl.kernel(
        body,
        out_type=jax.ShapeDtypeStruct((2 * n_acc,), jnp.float32),
        mesh=mesh,
        compiler_params=pltpu.CompilerParams(use_tc_tiling_on_sc=False),
        scratch_types=[
            pltpu.VMEM((C, _CHUNK), jnp.int32),
            pltpu.VMEM((_CHUNK,), jnp.float32),
            pltpu.VMEM_SHARED((n_acc,), jnp.float32),
            pltpu.SemaphoreType.DMA,
        ],
    )


@functools.lru_cache(None)
def _gather_kernel(n_src, m, n_out_pad):
    C = n_out_pad // _CHUNK // _NW

    mesh = plsc.VectorSubcoreMesh(core_axis_name="c", subcore_axis_name="s")

    def body(table, idx2d, out, idx_v, rows_v, sem):
        c = lax.axis_index("c")
        s = lax.axis_index("s")
        wid = c * 16 + s
        pltpu.sync_copy(idx2d.at[pl.ds(wid * C, C)], idx_v)

        def step(j, carry):
            pltpu.async_copy(table.at[idx_v.at[j]], rows_v, sem).wait()
            pltpu.sync_copy(rows_v, out.at[pl.ds((wid * C + j) * _CHUNK, _CHUNK)])
            return carry

        lax.fori_loop(0, C, step, 0)

    return pl.kernel(
        body,
        out_type=jax.ShapeDtypeStruct((n_out_pad, m), jnp.float32),
        mesh=mesh,
        compiler_params=pltpu.CompilerParams(use_tc_tiling_on_sc=False),
        scratch_types=[
            pltpu.VMEM((C, _CHUNK), jnp.int32),
            pltpu.VMEM((_CHUNK, m), jnp.float32),
            pltpu.SemaphoreType.DMA,
        ],
    )


def _gather_rows(table, idx, n_out):
    n_src, m = table.shape
    n_out_pad = _rup(n_out, _GRAIN)
    idx_p = jnp.concatenate([idx, jnp.zeros((n_out_pad - n_out,), jnp.int32)])
    out = _gather_kernel(n_src, m, n_out_pad)(table, idx_p.reshape(-1, _CHUNK))
    return out[:n_out]


def _norm(x, g, b, relu=True):
    mu = jnp.mean(x, axis=0, keepdims=True)
    v = jnp.var(x, axis=0, keepdims=True)
    y = (x - mu) / jnp.sqrt(v + 1e-5) * g + b
    return jax.nn.relu(y) if relu else y


def _head(x, p):
    h = _norm(x @ p['w1'], p['g'], p['b'])
    return h @ p['w2'] + p['b2']


def kernel(x, edge_index_0, edge_index_1, edge_index_2, edge_index_3,
           cluster_0, cluster_1, cluster_2, params):
    eis = [edge_index_0, edge_index_1, edge_index_2, edge_index_3]
    clusters = [cluster_0, cluster_1, cluster_2]

    e_pads = [_rup(e, _GRAIN) for e in _ES]
    srcs, dsts = [], []
    for i in range(4):
        ep = e_pads[i]
        src = jnp.concatenate([eis[i][0], jnp.zeros((ep - _ES[i],), jnp.int32)])
        dst = jnp.concatenate([eis[i][1],
                               jnp.full((ep - _ES[i],), _NS[i], jnp.int32)])
        srcs.append(src.reshape(-1, _CHUNK))
        dsts.append(dst.reshape(-1, _CHUNK))
    pool_pads = [_rup(n, _GRAIN) for n in _NS[:3]]
    pool_srcs, pool_dsts = [], []
    for i in range(3):
        pp = pool_pads[i]
        psrc = jnp.concatenate([jnp.arange(_NS[i], dtype=jnp.int32),
                                jnp.zeros((pp - _NS[i],), jnp.int32)])
        pdst = jnp.concatenate([clusters[i],
                                jnp.full((pp - _NS[i],), _NS[i + 1], jnp.int32)])
        pool_srcs.append(psrc.reshape(-1, _CHUNK))
        pool_dsts.append(pdst.reshape(-1, _CHUNK))

    sizes = _NS + _NS[1:]
    offs, tot = [], 0
    for sz in sizes:
        offs.append(tot)
        tot += sz
    hist_acc = _rup(tot + 1, 2048)
    parts = []
    for i in range(4):
        d = jnp.concatenate([eis[i][1] + offs[i],
                             jnp.full((e_pads[i] - _ES[i],), tot, jnp.int32)])
        parts.append(d)
    for i in range(3):
        d = jnp.concatenate([clusters[i] + offs[4 + i],
                             jnp.full((pool_pads[i] - _NS[i],), tot, jnp.int32)])
        parts.append(d)
    flat = jnp.concatenate(parts)
    hist_pad = _rup(flat.shape[0], _GRAIN)
    flat = jnp.concatenate([flat, jnp.full((hist_pad - flat.shape[0],), tot, jnp.int32)])
    hist_idx = flat.reshape(-1, _CHUNK)
    hk = _hist_kernel(hist_idx.shape[0], hist_acc)
    hout = hk(hist_idx, jnp.zeros((hist_acc,), jnp.float32))
    hist = hout[:hist_acc] + hout[hist_acc:]
    degs = [hist[offs[i]:offs[i] + _NS[i]] for i in range(4)]
    cnts = [hist[offs[4 + i]:offs[4 + i] + _NS[i + 1]] for i in range(3)]

    def gconv(h, lvl, w, b):
        agg = _seg_sum(h, srcs[lvl], dsts[lvl], _NS[lvl], e_pads[lvl])
        agg = agg / (degs[lvl] + 1.0)[:, None]
        return (h + agg) @ w + b

    def resblock(xx, lvl, p):
        h = _norm(xx @ p['w1'], p['g1'], p['b1'])
        h = _norm(gconv(h, lvl, p['wg'], p['bg']), p['g2'], p['b2'])
        h = _norm(h @ p['w2'], p['g3'], p['b3'], relu=False)
        return jax.nn.relu(xx + h)

    pc = params['conv1']
    y = x @ pc['w']
    agg = _seg_sum(y, srcs[0], dsts[0], _NS[0], e_pads[0])
    h = y + agg / (degs[0] + 1.0)[:, None] + pc['b']
    h = _norm(h, pc['g'], pc['bt'])

    convs = {}
    cur = h
    for i in range(4):
        for blk in params['enc'][i]:
            cur = resblock(cur, i, blk)
        convs[i] = cur
        if i < 3:
            s = _seg_sum(cur, pool_srcs[i], pool_dsts[i], _NS[i + 1], pool_pads[i])
            pooled = s / jnp.maximum(cnts[i], 1.0)[:, None]
            d = params['down'][i]
            cur = _norm(pooled @ d['w'], d['g'], d['bt'])

    deconv = convs[3]
    outs = []
    for i in range(4):
        L = 3 - i
        for blk in params['dec'][i]:
            deconv = resblock(deconv, L, blk)
        logit = _head(deconv, params['pred'][i])
        signal = jnp.tanh(_head(deconv, params['reg'][i]))
        outs.append(jnp.concatenate([logit, signal], axis=1))
        if i < 3:
            u = params['up'][i]
            t = deconv @ u['w']
            up = _gather_rows(t, clusters[L - 1], _NS[L - 1])
            deconv = _norm(up, u['g'], u['bt']) + convs[L - 1]
    return jnp.concatenate(outs, axis=0)


# speedup vs baseline: 9.8753x; 4.8534x over previous
import functools

import jax
import jax.numpy as jnp
from jax import lax
from jax.experimental import pallas as pl
from jax.experimental.pallas import tpu as pltpu
from jax.experimental.pallas import tpu_sc as plsc

jax.config.update("jax_default_matmul_precision", "float32")

_ENC_CH = [32, 32, 64, 128]
_DEC_CH = [128, 64, 32, 32]
_NS = [10000, 2500, 625, 156]
_ES = [320000, 80000, 20000, 5000]

_NW = 32
_CHUNK = 128
_GRAIN = _NW * 8 * _CHUNK
_NBUF = 4


def _rup(x, m):
    return (x + m - 1) // m * m


def _pad_spread(arr, n_pad, lo, hi):
    fill = lo + jnp.arange(n_pad, dtype=jnp.int32) % max(hi - lo, 1)
    return jnp.concatenate([arr, fill])


@functools.lru_cache(None)
def _seg_sum_kernel(n_src, m, n_dst, e_pad):
    C = e_pad // _CHUNK // _NW
    n_acc = _rup(n_dst + 1, 128)
    R = n_acc // 16
    mesh = plsc.VectorSubcoreMesh(core_axis_name="c", subcore_axis_name="s")

    nbuf = min(_NBUF, C)

    def body(values, src2d, dst2d, zeros, out, src_v, dst_v, rows_v, acc_sh,
             gsems, ssems):
        c = lax.axis_index("c")
        s = lax.axis_index("s")
        wid = c * 16 + s
        pltpu.sync_copy(zeros.at[pl.ds(s * R, R)], acc_sh.at[pl.ds(s * R, R)])
        pltpu.sync_copy(src2d.at[pl.ds(wid * C, C)], src_v)
        pltpu.sync_copy(dst2d.at[pl.ds(wid * C, C)], dst_v)
        plsc.subcore_barrier()

        gd = [None] * nbuf
        sd = [None] * nbuf
        for j in range(nbuf):
            gd[j] = pltpu.async_copy(values.at[src_v.at[j]], rows_v.at[j],
                                     gsems[j])
        for j in range(C):
            b = j % nbuf
            gd[b].wait()
            sd[b] = pltpu.async_copy(rows_v.at[b], acc_sh.at[dst_v.at[j]],
                                     ssems[b], add=True)
            nj = j + nbuf
            if nj < C:
                sd[b].wait()
                gd[b] = pltpu.async_copy(values.at[src_v.at[nj]],
                                         rows_v.at[b], gsems[b])
        for j in range(max(C - nbuf, 0), C):
            sd[j % nbuf].wait()
        plsc.subcore_barrier()
        pltpu.sync_copy(acc_sh.at[pl.ds(s * R, R)], out.at[c, pl.ds(s * R, R)])

    return pl.kernel(
        body,
        out_type=jax.ShapeDtypeStruct((2, n_acc, m), jnp.float32),
        mesh=mesh,
        compiler_params=pltpu.CompilerParams(use_tc_tiling_on_sc=False),
        scratch_types=[
            pltpu.VMEM((C, _CHUNK), jnp.int32),
            pltpu.VMEM((C, _CHUNK), jnp.int32),
            pltpu.VMEM((nbuf, _CHUNK, m), jnp.float32),
            pltpu.VMEM_SHARED((n_acc, m), jnp.float32),
            [pltpu.SemaphoreType.DMA] * nbuf,
            [pltpu.SemaphoreType.DMA] * nbuf,
        ],
    )


def _seg_sum(values, src2d, dst2d, n_dst, e_pad):
    n_src, m = values.shape
    k = _seg_sum_kernel(n_src, m, n_dst, e_pad)
    n_acc = _rup(n_dst + 1, 128)
    zeros = jnp.zeros((n_acc, m), jnp.float32)
    out = k(values, src2d, dst2d, zeros)
    return out[0, :n_dst] + out[1, :n_dst]


@functools.lru_cache(None)
def _hist_kernel(n_rows, n_acc):
    C = n_rows // _NW
    R = n_acc // 16
    mesh = plsc.VectorSubcoreMesh(core_axis_name="c", subcore_axis_name="s")

    nbuf = min(_NBUF, C)

    def body(dst2d, zeros, out, dst_v, ones_v, acc_sh, ssems):
        c = lax.axis_index("c")
        s = lax.axis_index("s")
        wid = c * 16 + s
        pltpu.sync_copy(zeros.at[pl.ds(s * R, R)], acc_sh.at[pl.ds(s * R, R)])
        pltpu.sync_copy(dst2d.at[pl.ds(wid * C, C)], dst_v)
        for i in range(_CHUNK // 16):
            ones_v[pl.ds(i * 16, 16)] = jnp.ones((16,), jnp.float32)
        plsc.subcore_barrier()

        sd = [None] * nbuf
        for j in range(C):
            b = j % nbuf
            if sd[b] is not None:
                sd[b].wait()
            sd[b] = pltpu.async_copy(ones_v, acc_sh.at[dst_v.at[j]],
                                     ssems[b], add=True)
        for b in range(nbuf):
            if sd[b] is not None:
                sd[b].wait()
        plsc.subcore_barrier()
        pltpu.sync_copy(acc_sh.at[pl.ds(s * R, R)],
                        out.at[pl.ds(c * n_acc + s * R, R)])

    return pl.kernel(
        body,
        out_type=jax.ShapeDtypeStruct((2 * n_acc,), jnp.float32),
        mesh=mesh,
        compiler_params=pltpu.CompilerParams(use_tc_tiling_on_sc=False),
        scratch_types=[
            pltpu.VMEM((C, _CHUNK), jnp.int32),
            pltpu.VMEM((_CHUNK,), jnp.float32),
            pltpu.VMEM_SHARED((n_acc,), jnp.float32),
            [pltpu.SemaphoreType.DMA] * nbuf,
        ],
    )


@functools.lru_cache(None)
def _gather_kernel(n_src, m, n_out_pad):
    C = n_out_pad // _CHUNK // _NW

    mesh = plsc.VectorSubcoreMesh(core_axis_name="c", subcore_axis_name="s")

    nbuf = min(_NBUF, C)

    def body(table, idx2d, out, idx_v, rows_v, gsems, wsems):
        c = lax.axis_index("c")
        s = lax.axis_index("s")
        wid = c * 16 + s
        pltpu.sync_copy(idx2d.at[pl.ds(wid * C, C)], idx_v)

        gd = [None] * nbuf
        wd = [None] * nbuf
        for j in range(nbuf):
            gd[j] = pltpu.async_copy(table.at[idx_v.at[j]], rows_v.at[j],
                                     gsems[j])
        for j in range(C):
            b = j % nbuf
            gd[b].wait()
            wd[b] = pltpu.async_copy(
                rows_v.at[b], out.at[pl.ds((wid * C + j) * _CHUNK, _CHUNK)],
                wsems[b])
            nj = j + nbuf
            if nj < C:
                wd[b].wait()
                gd[b] = pltpu.async_copy(table.at[idx_v.at[nj]],
                                         rows_v.at[b], gsems[b])
        for j in range(max(C - nbuf, 0), C):
            wd[j % nbuf].wait()

    return pl.kernel(
        body,
        out_type=jax.ShapeDtypeStruct((n_out_pad, m), jnp.float32),
        mesh=mesh,
        compiler_params=pltpu.CompilerParams(use_tc_tiling_on_sc=False),
        scratch_types=[
            pltpu.VMEM((C, _CHUNK), jnp.int32),
            pltpu.VMEM((nbuf, _CHUNK, m), jnp.float32),
            [pltpu.SemaphoreType.DMA] * nbuf,
            [pltpu.SemaphoreType.DMA] * nbuf,
        ],
    )


def _gather_rows(table, idx, n_out):
    n_src, m = table.shape
    n_out_pad = _rup(n_out, _GRAIN)
    idx_p = _pad_spread(idx, n_out_pad - n_out, 0, n_src)
    out = _gather_kernel(n_src, m, n_out_pad)(table, idx_p.reshape(-1, _CHUNK))
    return out[:n_out]


def _norm(x, g, b, relu=True):
    mu = jnp.mean(x, axis=0, keepdims=True)
    v = jnp.var(x, axis=0, keepdims=True)
    y = (x - mu) / jnp.sqrt(v + 1e-5) * g + b
    return jax.nn.relu(y) if relu else y


def _head(x, p):
    h = _norm(x @ p['w1'], p['g'], p['b'])
    return h @ p['w2'] + p['b2']


def kernel(x, edge_index_0, edge_index_1, edge_index_2, edge_index_3,
           cluster_0, cluster_1, cluster_2, params):
    eis = [edge_index_0, edge_index_1, edge_index_2, edge_index_3]
    clusters = [cluster_0, cluster_1, cluster_2]

    e_pads = [_rup(e, _GRAIN) for e in _ES]
    srcs, dsts = [], []
    for i in range(4):
        ep = e_pads[i]
        n_acc_i = _rup(_NS[i] + 1, 128)
        src = _pad_spread(eis[i][0], ep - _ES[i], 0, _NS[i])
        dst = _pad_spread(eis[i][1], ep - _ES[i], _NS[i], n_acc_i)
        srcs.append(src.reshape(-1, _CHUNK))
        dsts.append(dst.reshape(-1, _CHUNK))
    pool_pads = [_rup(n, _GRAIN) for n in _NS[:3]]
    pool_srcs, pool_dsts = [], []
    for i in range(3):
        pp = pool_pads[i]
        n_acc_i = _rup(_NS[i + 1] + 1, 128)
        psrc = _pad_spread(jnp.arange(_NS[i], dtype=jnp.int32),
                           pp - _NS[i], 0, _NS[i])
        pdst = _pad_spread(clusters[i], pp - _NS[i], _NS[i + 1], n_acc_i)
        pool_srcs.append(psrc.reshape(-1, _CHUNK))
        pool_dsts.append(pdst.reshape(-1, _CHUNK))

    sizes = _NS + _NS[1:]
    offs, tot = [], 0
    for sz in sizes:
        offs.append(tot)
        tot += sz
    hist_acc = _rup(tot + 1, 2048)
    parts = []
    for i in range(4):
        parts.append(_pad_spread(eis[i][1] + offs[i],
                                 e_pads[i] - _ES[i], tot, hist_acc))
    for i in range(3):
        parts.append(_pad_spread(clusters[i] + offs[4 + i],
                                 pool_pads[i] - _NS[i], tot, hist_acc))
    flat = jnp.concatenate(parts)
    hist_pad = _rup(flat.shape[0], _GRAIN)
    flat = _pad_spread(flat, hist_pad - flat.shape[0], tot, hist_acc)
    hist_idx = flat.reshape(-1, _CHUNK)
    hk = _hist_kernel(hist_idx.shape[0], hist_acc)
    hout = hk(hist_idx, jnp.zeros((hist_acc,), jnp.float32))
    hist = hout[:hist_acc] + hout[hist_acc:]
    degs = [hist[offs[i]:offs[i] + _NS[i]] for i in range(4)]
    cnts = [hist[offs[4 + i]:offs[4 + i] + _NS[i + 1]] for i in range(3)]

    def gconv(h, lvl, w, b):
        agg = _seg_sum(h, srcs[lvl], dsts[lvl], _NS[lvl], e_pads[lvl])
        agg = agg / (degs[lvl] + 1.0)[:, None]
        return (h + agg) @ w + b

    def resblock(xx, lvl, p):
        h = _norm(xx @ p['w1'], p['g1'], p['b1'])
        h = _norm(gconv(h, lvl, p['wg'], p['bg']), p['g2'], p['b2'])
        h = _norm(h @ p['w2'], p['g3'], p['b3'], relu=False)
        return jax.nn.relu(xx + h)

    pc = params['conv1']
    y = x @ pc['w']
    agg = _seg_sum(y, srcs[0], dsts[0], _NS[0], e_pads[0])
    h = y + agg / (degs[0] + 1.0)[:, None] + pc['b']
    h = _norm(h, pc['g'], pc['bt'])

    convs = {}
    cur = h
    for i in range(4):
        for blk in params['enc'][i]:
            cur = resblock(cur, i, blk)
        convs[i] = cur
        if i < 3:
            s = _seg_sum(cur, pool_srcs[i], pool_dsts[i], _NS[i + 1], pool_pads[i])
            pooled = s / jnp.maximum(cnts[i], 1.0)[:, None]
            d = params['down'][i]
            cur = _norm(pooled @ d['w'], d['g'], d['bt'])

    deconv = convs[3]
    outs = []
    for i in range(4):
        L = 3 - i
        for blk in params['dec'][i]:
            deconv = resblock(deconv, L, blk)
        logit = _head(deconv, params['pred'][i])
        signal = jnp.tanh(_head(deconv, params['reg'][i]))
        outs.append(jnp.concatenate([logit, signal], axis=1))
        if i < 3:
            u = params['up'][i]
            t = deconv @ u['w']
            up = _gather_rows(t, clusters[L - 1], _NS[L - 1])
            deconv = _norm(up, u['g'], u['bt']) + convs[L - 1]
    return jnp.concatenate(outs, axis=0)


# speedup vs baseline: 9.9526x; 1.0078x over previous
import functools

import jax
import jax.numpy as jnp
from jax import lax
from jax.experimental import pallas as pl
from jax.experimental.pallas import tpu as pltpu
from jax.experimental.pallas import tpu_sc as plsc

jax.config.update("jax_default_matmul_precision", "float32")

_ENC_CH = [32, 32, 64, 128]
_DEC_CH = [128, 64, 32, 32]
_NS = [10000, 2500, 625, 156]
_ES = [320000, 80000, 20000, 5000]

_NW = 32
_CHUNK = 128
_GRAIN = _NW * 8 * _CHUNK
_NBUF = 4


def _rup(x, m):
    return (x + m - 1) // m * m


def _pad_spread(arr, n_pad, lo, hi):
    fill = lo + jnp.arange(n_pad, dtype=jnp.int32) % max(hi - lo, 1)
    return jnp.concatenate([arr, fill])


@functools.lru_cache(None)
def _seg_sum_kernel(n_src, m, n_dst, e_pad):
    C = e_pad // _CHUNK // _NW
    n_acc = _rup(n_dst + 1, 128)
    R = n_acc // 16
    mesh = plsc.VectorSubcoreMesh(core_axis_name="c", subcore_axis_name="s")

    nbuf = min(_NBUF, C)

    def body(values, src2d, dst2d, zeros, out, src_v, dst_v, rows_v, acc_sh,
             gsems, ssems):
        c = lax.axis_index("c")
        s = lax.axis_index("s")
        wid = c * 16 + s
        pltpu.sync_copy(zeros.at[pl.ds(s * R, R)], acc_sh.at[pl.ds(s * R, R)])
        pltpu.sync_copy(src2d.at[pl.ds(wid * C, C)], src_v)
        pltpu.sync_copy(dst2d.at[pl.ds(wid * C, C)], dst_v)
        plsc.subcore_barrier()

        gd = [None] * nbuf
        sd = [None] * nbuf
        for j in range(nbuf):
            gd[j] = pltpu.async_copy(values.at[src_v.at[j]], rows_v.at[j],
                                     gsems[j])
        for j in range(C):
            b = j % nbuf
            gd[b].wait()
            sd[b] = pltpu.async_copy(rows_v.at[b], acc_sh.at[dst_v.at[j]],
                                     ssems[b], add=True)
            nj = j + nbuf
            if nj < C:
                sd[b].wait()
                gd[b] = pltpu.async_copy(values.at[src_v.at[nj]],
                                         rows_v.at[b], gsems[b])
        for j in range(max(C - nbuf, 0), C):
            sd[j % nbuf].wait()
        plsc.subcore_barrier()
        pltpu.sync_copy(acc_sh.at[pl.ds(s * R, R)], out.at[c, pl.ds(s * R, R)])

    return pl.kernel(
        body,
        out_type=jax.ShapeDtypeStruct((2, n_acc, m), jnp.float32),
        mesh=mesh,
        compiler_params=pltpu.CompilerParams(use_tc_tiling_on_sc=False),
        scratch_types=[
            pltpu.VMEM((C, _CHUNK), jnp.int32),
            pltpu.VMEM((C, _CHUNK), jnp.int32),
            pltpu.VMEM((nbuf, _CHUNK, m), jnp.float32),
            pltpu.VMEM_SHARED((n_acc, m), jnp.float32),
            [pltpu.SemaphoreType.DMA] * nbuf,
            [pltpu.SemaphoreType.DMA] * nbuf,
        ],
    )


def _seg_sum_p(values, src2d, dst2d, n_dst, e_pad):
    n_src, m = values.shape
    k = _seg_sum_kernel(n_src, m, n_dst, e_pad)
    n_acc = _rup(n_dst + 1, 128)
    zeros = jnp.zeros((n_acc, m), jnp.float32)
    return k(values, src2d, dst2d, zeros)


@functools.lru_cache(None)
def _hist_kernel(n_rows, n_acc):
    C = n_rows // _NW
    R = n_acc // 16
    mesh = plsc.VectorSubcoreMesh(core_axis_name="c", subcore_axis_name="s")

    nbuf = min(_NBUF, C)

    def body(dst2d, zeros, out, dst_v, ones_v, acc_sh, ssems):
        c = lax.axis_index("c")
        s = lax.axis_index("s")
        wid = c * 16 + s
        pltpu.sync_copy(zeros.at[pl.ds(s * R, R)], acc_sh.at[pl.ds(s * R, R)])
        pltpu.sync_copy(dst2d.at[pl.ds(wid * C, C)], dst_v)
        for i in range(_CHUNK // 16):
            ones_v[pl.ds(i * 16, 16)] = jnp.ones((16,), jnp.float32)
        plsc.subcore_barrier()

        sd = [None] * nbuf
        for j in range(C):
            b = j % nbuf
            if sd[b] is not None:
                sd[b].wait()
            sd[b] = pltpu.async_copy(ones_v, acc_sh.at[dst_v.at[j]],
                                     ssems[b], add=True)
        for b in range(nbuf):
            if sd[b] is not None:
                sd[b].wait()
        plsc.subcore_barrier()
        pltpu.sync_copy(acc_sh.at[pl.ds(s * R, R)],
                        out.at[pl.ds(c * n_acc + s * R, R)])

    return pl.kernel(
        body,
        out_type=jax.ShapeDtypeStruct((2 * n_acc,), jnp.float32),
        mesh=mesh,
        compiler_params=pltpu.CompilerParams(use_tc_tiling_on_sc=False),
        scratch_types=[
            pltpu.VMEM((C, _CHUNK), jnp.int32),
            pltpu.VMEM((_CHUNK,), jnp.float32),
            pltpu.VMEM_SHARED((n_acc,), jnp.float32),
            [pltpu.SemaphoreType.DMA] * nbuf,
        ],
    )


@functools.lru_cache(None)
def _gather_kernel(n_src, m, n_out_pad):
    C = n_out_pad // _CHUNK // _NW

    mesh = plsc.VectorSubcoreMesh(core_axis_name="c", subcore_axis_name="s")

    nbuf = min(_NBUF, C)

    def body(table, idx2d, out, idx_v, rows_v, gsems, wsems):
        c = lax.axis_index("c")
        s = lax.axis_index("s")
        wid = c * 16 + s
        pltpu.sync_copy(idx2d.at[pl.ds(wid * C, C)], idx_v)

        gd = [None] * nbuf
        wd = [None] * nbuf
        for j in range(nbuf):
            gd[j] = pltpu.async_copy(table.at[idx_v.at[j]], rows_v.at[j],
                                     gsems[j])
        for j in range(C):
            b = j % nbuf
            gd[b].wait()
            wd[b] = pltpu.async_copy(
                rows_v.at[b], out.at[pl.ds((wid * C + j) * _CHUNK, _CHUNK)],
                wsems[b])
            nj = j + nbuf
            if nj < C:
                wd[b].wait()
                gd[b] = pltpu.async_copy(table.at[idx_v.at[nj]],
                                         rows_v.at[b], gsems[b])
        for j in range(max(C - nbuf, 0), C):
            wd[j % nbuf].wait()

    return pl.kernel(
        body,
        out_type=jax.ShapeDtypeStruct((n_out_pad, m), jnp.float32),
        mesh=mesh,
        compiler_params=pltpu.CompilerParams(use_tc_tiling_on_sc=False),
        scratch_types=[
            pltpu.VMEM((C, _CHUNK), jnp.int32),
            pltpu.VMEM((nbuf, _CHUNK, m), jnp.float32),
            [pltpu.SemaphoreType.DMA] * nbuf,
            [pltpu.SemaphoreType.DMA] * nbuf,
        ],
    )


def _gather_rows(table, idx, n_out):
    n_src, m = table.shape
    n_out_pad = _rup(n_out, _GRAIN)
    idx_p = _pad_spread(idx, n_out_pad - n_out, 0, n_src)
    out = _gather_kernel(n_src, m, n_out_pad)(table, idx_p.reshape(-1, _CHUNK))
    return out[:n_out]


def _tc(fn, out_specs, *args):
    multi = isinstance(out_specs, (tuple, list))
    specs = tuple(out_specs) if multi else (out_specs,)
    n_in = len(args)

    def body(*refs):
        ins = tuple(r[...] for r in refs[:n_in])
        res = fn(*ins)
        res = res if isinstance(res, tuple) else (res,)
        for r, o in zip(refs[n_in:], res):
            r[...] = o

    out = pl.pallas_call(body, out_shape=specs)(*args)
    return out if multi else out[0]


def _norm(x, g, b, relu=True):
    mu = jnp.mean(x, axis=0, keepdims=True)
    v = jnp.var(x, axis=0, keepdims=True)
    y = (x - mu) / jnp.sqrt(v + 1e-5) * g + b
    return jax.nn.relu(y) if relu else y


def kernel(x, edge_index_0, edge_index_1, edge_index_2, edge_index_3,
           cluster_0, cluster_1, cluster_2, params):
    eis = [edge_index_0, edge_index_1, edge_index_2, edge_index_3]
    clusters = [cluster_0, cluster_1, cluster_2]

    e_pads = [_rup(e, _GRAIN) for e in _ES]
    srcs, dsts = [], []
    for i in range(4):
        ep = e_pads[i]
        n_acc_i = _rup(_NS[i] + 1, 128)
        src = _pad_spread(eis[i][0], ep - _ES[i], 0, _NS[i])
        dst = _pad_spread(eis[i][1], ep - _ES[i], _NS[i], n_acc_i)
        srcs.append(src.reshape(-1, _CHUNK))
        dsts.append(dst.reshape(-1, _CHUNK))
    pool_pads = [_rup(n, _GRAIN) for n in _NS[:3]]
    pool_srcs, pool_dsts = [], []
    for i in range(3):
        pp = pool_pads[i]
        n_acc_i = _rup(_NS[i + 1] + 1, 128)
        psrc = _pad_spread(jnp.arange(_NS[i], dtype=jnp.int32),
                           pp - _NS[i], 0, _NS[i])
        pdst = _pad_spread(clusters[i], pp - _NS[i], _NS[i + 1], n_acc_i)
        pool_srcs.append(psrc.reshape(-1, _CHUNK))
        pool_dsts.append(pdst.reshape(-1, _CHUNK))

    sizes = _NS + _NS[1:]
    offs, tot = [], 0
    for sz in sizes:
        offs.append(tot)
        tot += sz
    hist_acc = _rup(tot + 1, 2048)
    parts = []
    for i in range(4):
        parts.append(_pad_spread(eis[i][1] + offs[i],
                                 e_pads[i] - _ES[i], tot, hist_acc))
    for i in range(3):
        parts.append(_pad_spread(clusters[i] + offs[4 + i],
                                 pool_pads[i] - _NS[i], tot, hist_acc))
    flat = jnp.concatenate(parts)
    hist_pad = _rup(flat.shape[0], _GRAIN)
    flat = _pad_spread(flat, hist_pad - flat.shape[0], tot, hist_acc)
    hist_idx = flat.reshape(-1, _CHUNK)
    hk = _hist_kernel(hist_idx.shape[0], hist_acc)
    hout = hk(hist_idx, jnp.zeros((hist_acc,), jnp.float32))
    hist = hout[:hist_acc] + hout[hist_acc:]
    degs = [hist[offs[i]:offs[i] + _NS[i]] for i in range(4)]
    cnts = [hist[offs[4 + i]:offs[4 + i] + _NS[i + 1]] for i in range(3)]

    def S(*shape):
        return jax.ShapeDtypeStruct(shape, jnp.float32)

    def rs(v):
        return v.reshape(1, -1)

    def seg_p(vals, lvl):
        return _seg_sum_p(vals, srcs[lvl], dsts[lvl], _NS[lvl], e_pads[lvl])

    deg2 = [d.reshape(-1, 1) for d in degs]
    cnt2 = [c.reshape(-1, 1) for c in cnts]

    def pre_args(pre):
        return (pre['w1'], rs(pre['g1']), rs(pre['b1'])) if pre else ()

    def rb_post_stage(lvl, p, xx, h1, P, pre):
        n = _NS[lvl]
        c = xx.shape[1]

        def f(xx, h1, P, deg, wg, bg, g2, b2, w2, g3, b3, *nx):
            agg = (P[0, :n] + P[1, :n]) / (deg + 1.0)
            gg = (h1 + agg) @ wg + bg
            h2 = _norm(gg, g2, b2)
            u = _norm(h2 @ w2, g3, b3, relu=False)
            xo = jax.nn.relu(xx + u)
            if not nx:
                return xo
            w1n, g1n, b1n = nx
            return xo, _norm(xo @ w1n, g1n, b1n)

        outs = S(n, c) if not pre else [S(n, c), S(n, pre['w1'].shape[1])]
        return _tc(f, outs, xx, h1, P, deg2[lvl], p['wg'], rs(p['bg']),
                   rs(p['g2']), rs(p['b2']), p['w2'], rs(p['g3']), rs(p['b3']),
                   *pre_args(pre))

    def run_block(lvl, p, xx, h1, pre):
        P = seg_p(h1, lvl)
        return rb_post_stage(lvl, p, xx, h1, P, pre)

    pc = params['conv1']
    y = _tc(lambda xx, w: xx @ w, S(_NS[0], 32), x, pc['w'])
    P = seg_p(y, 0)
    blk0 = params['enc'][0][0]

    def conv1_post(y, P, deg, b, g, bt, w1n, g1n, b1n):
        agg = (P[0, :_NS[0]] + P[1, :_NS[0]]) / (deg + 1.0)
        h = _norm(y + agg + b, g, bt)
        return h, _norm(h @ w1n, g1n, b1n)

    cur, h1 = _tc(conv1_post, [S(_NS[0], 32), S(_NS[0], 8)],
                  y, P, deg2[0], rs(pc['b']), rs(pc['g']), rs(pc['bt']),
                  *pre_args(blk0))

    convs = {}
    for i in range(4):
        blocks = params['enc'][i]
        for bi, blk in enumerate(blocks):
            last_of_net_level = (bi == 2)
            if not last_of_net_level:
                pre = blocks[bi + 1]
            elif i < 3:
                pre = None
            else:
                pre = params['dec'][0][0]
            res = run_block(i, blk, cur, h1, pre)
            if pre:
                cur, h1 = res
            else:
                cur = res
        convs[i] = cur
        if i < 3:
            Pp = _seg_sum_p(cur, pool_srcs[i], pool_dsts[i],
                            _NS[i + 1], pool_pads[i])
            d = params['down'][i]
            nxt = params['enc'][i + 1][0]
            n1 = _NS[i + 1]

            def pool_post(P, cnt, dw, g, bt, w1n, g1n, b1n, n1=n1):
                pooled = (P[0, :n1] + P[1, :n1]) / jnp.maximum(cnt, 1.0)
                c2 = _norm(pooled @ dw, g, bt)
                return c2, _norm(c2 @ w1n, g1n, b1n)

            cur, h1 = _tc(pool_post,
                          [S(n1, d['w'].shape[1]), S(n1, nxt['w1'].shape[1])],
                          Pp, cnt2[i], d['w'], rs(d['g']), rs(d['bt']),
                          *pre_args(nxt))

    deconv = convs[3]
    outs = []
    for i in range(4):
        L = 3 - i
        blocks = params['dec'][i]
        for bi, blk in enumerate(blocks):
            if bi < 2:
                pre = blocks[bi + 1]
            else:
                pre = None
            res = run_block(L, blk, deconv, h1, pre)
            if pre:
                deconv, h1 = res
            else:
                deconv = res
        hp, hr = params['pred'][i], params['reg'][i]
        n = _NS[L]
        if i < 3:
            u = params['up'][i]

            def head_up(xx, pw1, pg, pb, pw2, pb2, rw1, rg, rb, rw2, rb2, uw):
                logit = _norm(xx @ pw1, pg, pb) @ pw2 + pb2
                signal = jnp.tanh(_norm(xx @ rw1, rg, rb) @ rw2 + rb2)
                return jnp.concatenate([logit, signal], axis=1), xx @ uw

            o_i, t = _tc(head_up, [S(n, 6), S(n, u['w'].shape[1])],
                         deconv, hp['w1'], rs(hp['g']), rs(hp['b']), hp['w2'],
                         rs(hp['b2']), hr['w1'], rs(hr['g']), rs(hr['b']),
                         hr['w2'], rs(hr['b2']), u['w'])
            outs.append(o_i)
            up = _gather_rows(t, clusters[L - 1], _NS[L - 1])
            nxt = params['dec'][i + 1][0]
            skip = convs[L - 1]
            n1 = _NS[L - 1]

            def up_post(up, sk, g, bt, w1n, g1n, b1n):
                dc = _norm(up, g, bt) + sk
                return dc, _norm(dc @ w1n, g1n, b1n)

            deconv, h1 = _tc(up_post,
                             [S(n1, skip.shape[1]), S(n1, nxt['w1'].shape[1])],
                             up, skip, rs(u['g']), rs(u['bt']), *pre_args(nxt))
        else:

            def head_only(xx, pw1, pg, pb, pw2, pb2, rw1, rg, rb, rw2, rb2):
                logit = _norm(xx @ pw1, pg, pb) @ pw2 + pb2
                signal = jnp.tanh(_norm(xx @ rw1, rg, rb) @ rw2 + rb2)
                return jnp.concatenate([logit, signal], axis=1)

            outs.append(_tc(head_only, S(n, 6),
                            deconv, hp['w1'], rs(hp['g']), rs(hp['b']),
                            hp['w2'], rs(hp['b2']), hr['w1'], rs(hr['g']),
                            rs(hr['b']), hr['w2'], rs(hr['b2'])))
    return jnp.concatenate(outs, axis=0)


# speedup vs baseline: 10.0087x; 1.0056x over previous
import functools

import jax
import jax.numpy as jnp
from jax import lax
from jax.experimental import pallas as pl
from jax.experimental.pallas import tpu as pltpu
from jax.experimental.pallas import tpu_sc as plsc

jax.config.update("jax_default_matmul_precision", "float32")

_ENC_CH = [32, 32, 64, 128]
_DEC_CH = [128, 64, 32, 32]
_NS = [10000, 2500, 625, 156]
_ES = [320000, 80000, 20000, 5000]

_NW = 32
_CHUNK = 128
_GRAIN = _NW * 8 * _CHUNK
_NBUF = 4


def _rup(x, m):
    return (x + m - 1) // m * m


def _pad_spread(arr, n_pad, lo, hi):
    fill = lo + jnp.arange(n_pad, dtype=jnp.int32) % max(hi - lo, 1)
    return jnp.concatenate([arr, fill])


@functools.lru_cache(None)
def _seg_sum_kernel(n_src, m, n_dst, e_pad):
    C = e_pad // _CHUNK // _NW
    n_acc = _rup(n_dst + 1, 128)
    R = n_acc // 16
    mesh = plsc.VectorSubcoreMesh(core_axis_name="c", subcore_axis_name="s")

    nbuf = min(_NBUF, C)

    def body(values, src2d, dst2d, zeros, out, src_v, dst_v, rows_v, acc_sh,
             gsems, ssems):
        c = lax.axis_index("c")
        s = lax.axis_index("s")
        wid = c * 16 + s
        pltpu.sync_copy(zeros.at[pl.ds(s * R, R)], acc_sh.at[pl.ds(s * R, R)])
        pltpu.sync_copy(src2d.at[pl.ds(wid * C, C)], src_v)
        pltpu.sync_copy(dst2d.at[pl.ds(wid * C, C)], dst_v)
        plsc.subcore_barrier()

        G = C // nbuf
        for b in range(nbuf):
            pltpu.async_copy(values.at[src_v.at[b]], rows_v.at[b], gsems[b])

        def grp(g, carry):
            base = g * nbuf
            for b in range(nbuf):
                j = base + b
                pltpu.make_async_copy(values.at[src_v.at[j]], rows_v.at[b],
                                      gsems[b]).wait()
                pltpu.async_copy(rows_v.at[b], acc_sh.at[dst_v.at[j]],
                                 ssems[b], add=True)

            @pl.when(g < G - 1)
            def _():
                for b in range(nbuf):
                    j = base + b
                    pltpu.make_async_copy(rows_v.at[b],
                                          acc_sh.at[dst_v.at[j]],
                                          ssems[b]).wait()
                    pltpu.async_copy(values.at[src_v.at[j + nbuf]],
                                     rows_v.at[b], gsems[b])

            return carry

        lax.fori_loop(0, G, grp, 0)
        for b in range(nbuf):
            pltpu.make_async_copy(rows_v.at[b], acc_sh.at[dst_v.at[b]],
                                  ssems[b]).wait()
        plsc.subcore_barrier()
        pltpu.sync_copy(acc_sh.at[pl.ds(s * R, R)], out.at[c, pl.ds(s * R, R)])

    return pl.kernel(
        body,
        out_type=jax.ShapeDtypeStruct((2, n_acc, m), jnp.float32),
        mesh=mesh,
        compiler_params=pltpu.CompilerParams(use_tc_tiling_on_sc=False),
        scratch_types=[
            pltpu.VMEM((C, _CHUNK), jnp.int32),
            pltpu.VMEM((C, _CHUNK), jnp.int32),
            pltpu.VMEM((nbuf, _CHUNK, m), jnp.float32),
            pltpu.VMEM_SHARED((n_acc, m), jnp.float32),
            [pltpu.SemaphoreType.DMA] * nbuf,
            [pltpu.SemaphoreType.DMA] * nbuf,
        ],
    )


def _seg_sum_p(values, src2d, dst2d, n_dst, e_pad):
    n_src, m = values.shape
    k = _seg_sum_kernel(n_src, m, n_dst, e_pad)
    n_acc = _rup(n_dst + 1, 128)
    zeros = jnp.zeros((n_acc, m), jnp.float32)
    return k(values, src2d, dst2d, zeros)


@functools.lru_cache(None)
def _hist_kernel(n_rows, n_acc):
    C = n_rows // _NW
    R = n_acc // 16
    mesh = plsc.VectorSubcoreMesh(core_axis_name="c", subcore_axis_name="s")

    nbuf = min(_NBUF, C)

    def body(dst2d, zeros, out, dst_v, ones_v, acc_sh, ssems):
        c = lax.axis_index("c")
        s = lax.axis_index("s")
        wid = c * 16 + s
        pltpu.sync_copy(zeros.at[pl.ds(s * R, R)], acc_sh.at[pl.ds(s * R, R)])
        pltpu.sync_copy(dst2d.at[pl.ds(wid * C, C)], dst_v)
        for i in range(_CHUNK // 16):
            ones_v[pl.ds(i * 16, 16)] = jnp.ones((16,), jnp.float32)
        plsc.subcore_barrier()

        G = C // nbuf

        def grp(g, carry):
            base = g * nbuf

            @pl.when(g > 0)
            def _():
                for b in range(nbuf):
                    pltpu.make_async_copy(ones_v, acc_sh.at[dst_v.at[b]],
                                          ssems[b]).wait()

            for b in range(nbuf):
                pltpu.async_copy(ones_v, acc_sh.at[dst_v.at[base + b]],
                                 ssems[b], add=True)
            return carry

        lax.fori_loop(0, G, grp, 0)
        for b in range(nbuf):
            pltpu.make_async_copy(ones_v, acc_sh.at[dst_v.at[b]],
                                  ssems[b]).wait()
        plsc.subcore_barrier()
        pltpu.sync_copy(acc_sh.at[pl.ds(s * R, R)],
                        out.at[pl.ds(c * n_acc + s * R, R)])

    return pl.kernel(
        body,
        out_type=jax.ShapeDtypeStruct((2 * n_acc,), jnp.float32),
        mesh=mesh,
        compiler_params=pltpu.CompilerParams(use_tc_tiling_on_sc=False),
        scratch_types=[
            pltpu.VMEM((C, _CHUNK), jnp.int32),
            pltpu.VMEM((_CHUNK,), jnp.float32),
            pltpu.VMEM_SHARED((n_acc,), jnp.float32),
            [pltpu.SemaphoreType.DMA] * nbuf,
        ],
    )


@functools.lru_cache(None)
def _gather_kernel(n_src, m, n_out_pad):
    C = n_out_pad // _CHUNK // _NW

    mesh = plsc.VectorSubcoreMesh(core_axis_name="c", subcore_axis_name="s")

    nbuf = min(_NBUF, C)

    def body(table, idx2d, out, idx_v, rows_v, gsems, wsems):
        c = lax.axis_index("c")
        s = lax.axis_index("s")
        wid = c * 16 + s
        pltpu.sync_copy(idx2d.at[pl.ds(wid * C, C)], idx_v)

        gd = [None] * nbuf
        wd = [None] * nbuf
        for j in range(nbuf):
            gd[j] = pltpu.async_copy(table.at[idx_v.at[j]], rows_v.at[j],
                                     gsems[j])
        for j in range(C):
            b = j % nbuf
            gd[b].wait()
            wd[b] = pltpu.async_copy(
                rows_v.at[b], out.at[pl.ds((wid * C + j) * _CHUNK, _CHUNK)],
                wsems[b])
            nj = j + nbuf
            if nj < C:
                wd[b].wait()
                gd[b] = pltpu.async_copy(table.at[idx_v.at[nj]],
                                         rows_v.at[b], gsems[b])
        for j in range(max(C - nbuf, 0), C):
            wd[j % nbuf].wait()

    return pl.kernel(
        body,
        out_type=jax.ShapeDtypeStruct((n_out_pad, m), jnp.float32),
        mesh=mesh,
        compiler_params=pltpu.CompilerParams(use_tc_tiling_on_sc=False),
        scratch_types=[
            pltpu.VMEM((C, _CHUNK), jnp.int32),
            pltpu.VMEM((nbuf, _CHUNK, m), jnp.float32),
            [pltpu.SemaphoreType.DMA] * nbuf,
            [pltpu.SemaphoreType.DMA] * nbuf,
        ],
    )


def _gather_rows(table, idx, n_out):
    n_src, m = table.shape
    n_out_pad = _rup(n_out, _GRAIN)
    idx_p = _pad_spread(idx, n_out_pad - n_out, 0, n_src)
    out = _gather_kernel(n_src, m, n_out_pad)(table, idx_p.reshape(-1, _CHUNK))
    return out[:n_out]


def _tc(fn, out_specs, *args):
    multi = isinstance(out_specs, (tuple, list))
    specs = tuple(out_specs) if multi else (out_specs,)
    n_in = len(args)

    def body(*refs):
        ins = tuple(r[...] for r in refs[:n_in])
        res = fn(*ins)
        res = res if isinstance(res, tuple) else (res,)
        for r, o in zip(refs[n_in:], res):
            r[...] = o

    out = pl.pallas_call(body, out_shape=specs)(*args)
    return out if multi else out[0]


def _norm(x, g, b, relu=True):
    mu = jnp.mean(x, axis=0, keepdims=True)
    v = jnp.var(x, axis=0, keepdims=True)
    y = (x - mu) / jnp.sqrt(v + 1e-5) * g + b
    return jax.nn.relu(y) if relu else y


def kernel(x, edge_index_0, edge_index_1, edge_index_2, edge_index_3,
           cluster_0, cluster_1, cluster_2, params):
    eis = [edge_index_0, edge_index_1, edge_index_2, edge_index_3]
    clusters = [cluster_0, cluster_1, cluster_2]

    e_pads = [_rup(e, _GRAIN) for e in _ES]
    srcs, dsts = [], []
    for i in range(4):
        ep = e_pads[i]
        n_acc_i = _rup(_NS[i] + 1, 128)
        src = _pad_spread(eis[i][0], ep - _ES[i], 0, _NS[i])
        dst = _pad_spread(eis[i][1], ep - _ES[i], _NS[i], n_acc_i)
        srcs.append(src.reshape(-1, _CHUNK))
        dsts.append(dst.reshape(-1, _CHUNK))
    pool_pads = [_rup(n, _GRAIN) for n in _NS[:3]]
    pool_srcs, pool_dsts = [], []
    for i in range(3):
        pp = pool_pads[i]
        n_acc_i = _rup(_NS[i + 1] + 1, 128)
        psrc = _pad_spread(jnp.arange(_NS[i], dtype=jnp.int32),
                           pp - _NS[i], 0, _NS[i])
        pdst = _pad_spread(clusters[i], pp - _NS[i], _NS[i + 1], n_acc_i)
        pool_srcs.append(psrc.reshape(-1, _CHUNK))
        pool_dsts.append(pdst.reshape(-1, _CHUNK))

    sizes = _NS + _NS[1:]
    offs, tot = [], 0
    for sz in sizes:
        offs.append(tot)
        tot += sz
    hist_acc = _rup(tot + 1, 2048)
    parts = []
    for i in range(4):
        parts.append(_pad_spread(eis[i][1] + offs[i],
                                 e_pads[i] - _ES[i], tot, hist_acc))
    for i in range(3):
        parts.append(_pad_spread(clusters[i] + offs[4 + i],
                                 pool_pads[i] - _NS[i], tot, hist_acc))
    flat = jnp.concatenate(parts)
    hist_pad = _rup(flat.shape[0], _GRAIN)
    flat = _pad_spread(flat, hist_pad - flat.shape[0], tot, hist_acc)
    hist_idx = flat.reshape(-1, _CHUNK)
    hk = _hist_kernel(hist_idx.shape[0], hist_acc)
    hout = hk(hist_idx, jnp.zeros((hist_acc,), jnp.float32))
    hist = hout[:hist_acc] + hout[hist_acc:]
    degs = [hist[offs[i]:offs[i] + _NS[i]] for i in range(4)]
    cnts = [hist[offs[4 + i]:offs[4 + i] + _NS[i + 1]] for i in range(3)]

    def S(*shape):
        return jax.ShapeDtypeStruct(shape, jnp.float32)

    def rs(v):
        return v.reshape(1, -1)

    def seg_p(vals, lvl):
        return _seg_sum_p(vals, srcs[lvl], dsts[lvl], _NS[lvl], e_pads[lvl])

    deg2 = [d.reshape(-1, 1) for d in degs]
    cnt2 = [c.reshape(-1, 1) for c in cnts]

    def pre_args(pre):
        return (pre['w1'], rs(pre['g1']), rs(pre['b1'])) if pre else ()

    def rb_post_stage(lvl, p, xx, h1, P, pre):
        n = _NS[lvl]
        c = xx.shape[1]

        def f(xx, h1, P, deg, wg, bg, g2, b2, w2, g3, b3, *nx):
            agg = (P[0, :n] + P[1, :n]) / (deg + 1.0)
            gg = (h1 + agg) @ wg + bg
            h2 = _norm(gg, g2, b2)
            u = _norm(h2 @ w2, g3, b3, relu=False)
            xo = jax.nn.relu(xx + u)
            if not nx:
                return xo
            w1n, g1n, b1n = nx
            return xo, _norm(xo @ w1n, g1n, b1n)

        outs = S(n, c) if not pre else [S(n, c), S(n, pre['w1'].shape[1])]
        return _tc(f, outs, xx, h1, P, deg2[lvl], p['wg'], rs(p['bg']),
                   rs(p['g2']), rs(p['b2']), p['w2'], rs(p['g3']), rs(p['b3']),
                   *pre_args(pre))

    def run_block(lvl, p, xx, h1, pre):
        P = seg_p(h1, lvl)
        return rb_post_stage(lvl, p, xx, h1, P, pre)

    pc = params['conv1']
    y = _tc(lambda xx, w: xx @ w, S(_NS[0], 32), x, pc['w'])
    P = seg_p(y, 0)
    blk0 = params['enc'][0][0]

    def conv1_post(y, P, deg, b, g, bt, w1n, g1n, b1n):
        agg = (P[0, :_NS[0]] + P[1, :_NS[0]]) / (deg + 1.0)
        h = _norm(y + agg + b, g, bt)
        return h, _norm(h @ w1n, g1n, b1n)

    cur, h1 = _tc(conv1_post, [S(_NS[0], 32), S(_NS[0], 8)],
                  y, P, deg2[0], rs(pc['b']), rs(pc['g']), rs(pc['bt']),
                  *pre_args(blk0))

    convs = {}
    for i in range(4):
        blocks = params['enc'][i]
        for bi, blk in enumerate(blocks):
            last_of_net_level = (bi == 2)
            if not last_of_net_level:
                pre = blocks[bi + 1]
            elif i < 3:
                pre = None
            else:
                pre = params['dec'][0][0]
            res = run_block(i, blk, cur, h1, pre)
            if pre:
                cur, h1 = res
            else:
                cur = res
        convs[i] = cur
        if i < 3:
            Pp = _seg_sum_p(cur, pool_srcs[i], pool_dsts[i],
                            _NS[i + 1], pool_pads[i])
            d = params['down'][i]
            nxt = params['enc'][i + 1][0]
            n1 = _NS[i + 1]

            def pool_post(P, cnt, dw, g, bt, w1n, g1n, b1n, n1=n1):
                pooled = (P[0, :n1] + P[1, :n1]) / jnp.maximum(cnt, 1.0)
                c2 = _norm(pooled @ dw, g, bt)
                return c2, _norm(c2 @ w1n, g1n, b1n)

            cur, h1 = _tc(pool_post,
                          [S(n1, d['w'].shape[1]), S(n1, nxt['w1'].shape[1])],
                          Pp, cnt2[i], d['w'], rs(d['g']), rs(d['bt']),
                          *pre_args(nxt))

    deconv = convs[3]
    outs = []
    for i in range(4):
        L = 3 - i
        blocks = params['dec'][i]
        for bi, blk in enumerate(blocks):
            if bi < 2:
                pre = blocks[bi + 1]
            else:
                pre = None
            res = run_block(L, blk, deconv, h1, pre)
            if pre:
                deconv, h1 = res
            else:
                deconv = res
        hp, hr = params['pred'][i], params['reg'][i]
        n = _NS[L]
        if i < 3:
            u = params['up'][i]

            def head_up(xx, pw1, pg, pb, pw2, pb2, rw1, rg, rb, rw2, rb2, uw):
                logit = _norm(xx @ pw1, pg, pb) @ pw2 + pb2
                signal = jnp.tanh(_norm(xx @ rw1, rg, rb) @ rw2 + rb2)
                return jnp.concatenate([logit, signal], axis=1), xx @ uw

            o_i, t = _tc(head_up, [S(n, 6), S(n, u['w'].shape[1])],
                         deconv, hp['w1'], rs(hp['g']), rs(hp['b']), hp['w2'],
                         rs(hp['b2']), hr['w1'], rs(hr['g']), rs(hr['b']),
                         hr['w2'], rs(hr['b2']), u['w'])
            outs.append(o_i)
            up = _gather_rows(t, clusters[L - 1], _NS[L - 1])
            nxt = params['dec'][i + 1][0]
            skip = convs[L - 1]
            n1 = _NS[L - 1]

            def up_post(up, sk, g, bt, w1n, g1n, b1n):
                dc = _norm(up, g, bt) + sk
                return dc, _norm(dc @ w1n, g1n, b1n)

            deconv, h1 = _tc(up_post,
                             [S(n1, skip.shape[1]), S(n1, nxt['w1'].shape[1])],
                             up, skip, rs(u['g']), rs(u['bt']), *pre_args(nxt))
        else:

            def head_only(xx, pw1, pg, pb, pw2, pb2, rw1, rg, rb, rw2, rb2):
                logit = _norm(xx @ pw1, pg, pb) @ pw2 + pb2
                signal = jnp.tanh(_norm(xx @ rw1, rg, rb) @ rw2 + rb2)
                return jnp.concatenate([logit, signal], axis=1)

            outs.append(_tc(head_only, S(n, 6),
                            deconv, hp['w1'], rs(hp['g']), rs(hp['b']),
                            hp['w2'], rs(hp['b2']), hr['w1'], rs(hr['g']),
                            rs(hr['b']), hr['w2'], rs(hr['b2'])))
    return jnp.concatenate(outs, axis=0)


# speedup vs baseline: 10.3299x; 1.0321x over previous
import functools

import jax
import jax.numpy as jnp
from jax import lax
from jax.experimental import pallas as pl
from jax.experimental.pallas import tpu as pltpu
from jax.experimental.pallas import tpu_sc as plsc

jax.config.update("jax_default_matmul_precision", "float32")

_ENC_CH = [32, 32, 64, 128]
_DEC_CH = [128, 64, 32, 32]
_NS = [10000, 2500, 625, 156]
_ES = [320000, 80000, 20000, 5000]

_NW = 32
_CHUNK = 128
_GRAIN = _NW * 8 * _CHUNK
_NBUF = 8


def _rup(x, m):
    return (x + m - 1) // m * m


def _pad_spread(arr, n_pad, lo, hi):
    fill = lo + jnp.arange(n_pad, dtype=jnp.int32) % max(hi - lo, 1)
    return jnp.concatenate([arr, fill])


@functools.lru_cache(None)
def _seg_sum_kernel(n_src, m, n_dst, e_pad):
    C = e_pad // _CHUNK // _NW
    n_acc = _rup(n_dst + 1, 128)
    R = n_acc // 16
    mesh = plsc.VectorSubcoreMesh(core_axis_name="c", subcore_axis_name="s")

    nbuf = min(_NBUF, C)

    def body(values, src2d, dst2d, zeros, out, src_v, dst_v, rows_v, acc_sh,
             gsems, ssems):
        c = lax.axis_index("c")
        s = lax.axis_index("s")
        wid = c * 16 + s
        pltpu.sync_copy(zeros.at[pl.ds(s * R, R)], acc_sh.at[pl.ds(s * R, R)])
        pltpu.sync_copy(src2d.at[pl.ds(wid * C, C)], src_v)
        pltpu.sync_copy(dst2d.at[pl.ds(wid * C, C)], dst_v)
        plsc.subcore_barrier()

        G = C // nbuf
        for b in range(nbuf):
            pltpu.async_copy(values.at[src_v.at[b]], rows_v.at[b], gsems[b])

        def grp(g, carry):
            base = g * nbuf
            for b in range(nbuf):
                j = base + b
                pltpu.make_async_copy(values.at[src_v.at[j]], rows_v.at[b],
                                      gsems[b]).wait()
                pltpu.async_copy(rows_v.at[b], acc_sh.at[dst_v.at[j]],
                                 ssems[b], add=True)

            @pl.when(g < G - 1)
            def _():
                for b in range(nbuf):
                    j = base + b
                    pltpu.make_async_copy(rows_v.at[b],
                                          acc_sh.at[dst_v.at[j]],
                                          ssems[b]).wait()
                    pltpu.async_copy(values.at[src_v.at[j + nbuf]],
                                     rows_v.at[b], gsems[b])

            return carry

        lax.fori_loop(0, G, grp, 0)
        for b in range(nbuf):
            pltpu.make_async_copy(rows_v.at[b], acc_sh.at[dst_v.at[b]],
                                  ssems[b]).wait()
        plsc.subcore_barrier()
        pltpu.sync_copy(acc_sh.at[pl.ds(s * R, R)], out.at[c, pl.ds(s * R, R)])

    return pl.kernel(
        body,
        out_type=jax.ShapeDtypeStruct((2, n_acc, m), jnp.float32),
        mesh=mesh,
        compiler_params=pltpu.CompilerParams(use_tc_tiling_on_sc=False),
        scratch_types=[
            pltpu.VMEM((C, _CHUNK), jnp.int32),
            pltpu.VMEM((C, _CHUNK), jnp.int32),
            pltpu.VMEM((nbuf, _CHUNK, m), jnp.float32),
            pltpu.VMEM_SHARED((n_acc, m), jnp.float32),
            [pltpu.SemaphoreType.DMA] * nbuf,
            [pltpu.SemaphoreType.DMA] * nbuf,
        ],
    )


def _seg_sum_p(values, src2d, dst2d, n_dst, e_pad):
    n_src, m = values.shape
    k = _seg_sum_kernel(n_src, m, n_dst, e_pad)
    n_acc = _rup(n_dst + 1, 128)
    zeros = jnp.zeros((n_acc, m), jnp.float32)
    return k(values, src2d, dst2d, zeros)


@functools.lru_cache(None)
def _hist_kernel(n_rows, n_acc):
    C = n_rows // _NW
    R = n_acc // 16
    mesh = plsc.VectorSubcoreMesh(core_axis_name="c", subcore_axis_name="s")

    nbuf = min(_NBUF, C)

    def body(dst2d, zeros, out, dst_v, ones_v, acc_sh, ssems):
        c = lax.axis_index("c")
        s = lax.axis_index("s")
        wid = c * 16 + s
        pltpu.sync_copy(zeros.at[pl.ds(s * R, R)], acc_sh.at[pl.ds(s * R, R)])
        pltpu.sync_copy(dst2d.at[pl.ds(wid * C, C)], dst_v)
        for i in range(_CHUNK // 16):
            ones_v[pl.ds(i * 16, 16)] = jnp.ones((16,), jnp.float32)
        plsc.subcore_barrier()

        G = C // nbuf

        def grp(g, carry):
            base = g * nbuf

            @pl.when(g > 0)
            def _():
                for b in range(nbuf):
                    pltpu.make_async_copy(ones_v, acc_sh.at[dst_v.at[b]],
                                          ssems[b]).wait()

            for b in range(nbuf):
                pltpu.async_copy(ones_v, acc_sh.at[dst_v.at[base + b]],
                                 ssems[b], add=True)
            return carry

        lax.fori_loop(0, G, grp, 0)
        for b in range(nbuf):
            pltpu.make_async_copy(ones_v, acc_sh.at[dst_v.at[b]],
                                  ssems[b]).wait()
        plsc.subcore_barrier()
        pltpu.sync_copy(acc_sh.at[pl.ds(s * R, R)],
                        out.at[pl.ds(c * n_acc + s * R, R)])

    return pl.kernel(
        body,
        out_type=jax.ShapeDtypeStruct((2 * n_acc,), jnp.float32),
        mesh=mesh,
        compiler_params=pltpu.CompilerParams(use_tc_tiling_on_sc=False),
        scratch_types=[
            pltpu.VMEM((C, _CHUNK), jnp.int32),
            pltpu.VMEM((_CHUNK,), jnp.float32),
            pltpu.VMEM_SHARED((n_acc,), jnp.float32),
            [pltpu.SemaphoreType.DMA] * nbuf,
        ],
    )


@functools.lru_cache(None)
def _gather_kernel(n_src, m, n_out_pad):
    C = n_out_pad // _CHUNK // _NW

    mesh = plsc.VectorSubcoreMesh(core_axis_name="c", subcore_axis_name="s")

    nbuf = min(_NBUF, C)

    def body(table, idx2d, out, idx_v, rows_v, gsems, wsems):
        c = lax.axis_index("c")
        s = lax.axis_index("s")
        wid = c * 16 + s
        pltpu.sync_copy(idx2d.at[pl.ds(wid * C, C)], idx_v)

        gd = [None] * nbuf
        wd = [None] * nbuf
        for j in range(nbuf):
            gd[j] = pltpu.async_copy(table.at[idx_v.at[j]], rows_v.at[j],
                                     gsems[j])
        for j in range(C):
            b = j % nbuf
            gd[b].wait()
            wd[b] = pltpu.async_copy(
                rows_v.at[b], out.at[pl.ds((wid * C + j) * _CHUNK, _CHUNK)],
                wsems[b])
            nj = j + nbuf
            if nj < C:
                wd[b].wait()
                gd[b] = pltpu.async_copy(table.at[idx_v.at[nj]],
                                         rows_v.at[b], gsems[b])
        for j in range(max(C - nbuf, 0), C):
            wd[j % nbuf].wait()

    return pl.kernel(
        body,
        out_type=jax.ShapeDtypeStruct((n_out_pad, m), jnp.float32),
        mesh=mesh,
        compiler_params=pltpu.CompilerParams(use_tc_tiling_on_sc=False),
        scratch_types=[
            pltpu.VMEM((C, _CHUNK), jnp.int32),
            pltpu.VMEM((nbuf, _CHUNK, m), jnp.float32),
            [pltpu.SemaphoreType.DMA] * nbuf,
            [pltpu.SemaphoreType.DMA] * nbuf,
        ],
    )


def _gather_rows(table, idx, n_out):
    n_src, m = table.shape
    n_out_pad = _rup(n_out, _GRAIN)
    idx_p = _pad_spread(idx, n_out_pad - n_out, 0, n_src)
    out = _gather_kernel(n_src, m, n_out_pad)(table, idx_p.reshape(-1, _CHUNK))
    return out[:n_out]


def _tc(fn, out_specs, *args):
    multi = isinstance(out_specs, (tuple, list))
    specs = tuple(out_specs) if multi else (out_specs,)
    n_in = len(args)

    def body(*refs):
        ins = tuple(r[...] for r in refs[:n_in])
        res = fn(*ins)
        res = res if isinstance(res, tuple) else (res,)
        for r, o in zip(refs[n_in:], res):
            r[...] = o

    out = pl.pallas_call(body, out_shape=specs)(*args)
    return out if multi else out[0]


def _norm(x, g, b, relu=True):
    mu = jnp.mean(x, axis=0, keepdims=True)
    v = jnp.var(x, axis=0, keepdims=True)
    y = (x - mu) / jnp.sqrt(v + 1e-5) * g + b
    return jax.nn.relu(y) if relu else y


def kernel(x, edge_index_0, edge_index_1, edge_index_2, edge_index_3,
           cluster_0, cluster_1, cluster_2, params):
    eis = [edge_index_0, edge_index_1, edge_index_2, edge_index_3]
    clusters = [cluster_0, cluster_1, cluster_2]

    e_pads = [_rup(e, _GRAIN) for e in _ES]
    srcs, dsts = [], []
    for i in range(4):
        ep = e_pads[i]
        n_acc_i = _rup(_NS[i] + 1, 128)
        src = _pad_spread(eis[i][0], ep - _ES[i], 0, _NS[i])
        dst = _pad_spread(eis[i][1], ep - _ES[i], _NS[i], n_acc_i)
        srcs.append(src.reshape(-1, _CHUNK))
        dsts.append(dst.reshape(-1, _CHUNK))
    pool_pads = [_rup(n, _GRAIN) for n in _NS[:3]]
    pool_srcs, pool_dsts = [], []
    for i in range(3):
        pp = pool_pads[i]
        n_acc_i = _rup(_NS[i + 1] + 1, 128)
        psrc = _pad_spread(jnp.arange(_NS[i], dtype=jnp.int32),
                           pp - _NS[i], 0, _NS[i])
        pdst = _pad_spread(clusters[i], pp - _NS[i], _NS[i + 1], n_acc_i)
        pool_srcs.append(psrc.reshape(-1, _CHUNK))
        pool_dsts.append(pdst.reshape(-1, _CHUNK))

    sizes = _NS + _NS[1:]
    offs, tot = [], 0
    for sz in sizes:
        offs.append(tot)
        tot += sz
    hist_acc = _rup(tot + 1, 2048)
    parts = []
    for i in range(4):
        parts.append(_pad_spread(eis[i][1] + offs[i],
                                 e_pads[i] - _ES[i], tot, hist_acc))
    for i in range(3):
        parts.append(_pad_spread(clusters[i] + offs[4 + i],
                                 pool_pads[i] - _NS[i], tot, hist_acc))
    flat = jnp.concatenate(parts)
    hist_pad = _rup(flat.shape[0], _GRAIN)
    flat = _pad_spread(flat, hist_pad - flat.shape[0], tot, hist_acc)
    hist_idx = flat.reshape(-1, _CHUNK)
    hk = _hist_kernel(hist_idx.shape[0], hist_acc)
    hout = hk(hist_idx, jnp.zeros((hist_acc,), jnp.float32))
    hist = hout[:hist_acc] + hout[hist_acc:]
    degs = [hist[offs[i]:offs[i] + _NS[i]] for i in range(4)]
    cnts = [hist[offs[4 + i]:offs[4 + i] + _NS[i + 1]] for i in range(3)]

    def S(*shape):
        return jax.ShapeDtypeStruct(shape, jnp.float32)

    def rs(v):
        return v.reshape(1, -1)

    def seg_p(vals, lvl):
        return _seg_sum_p(vals, srcs[lvl], dsts[lvl], _NS[lvl], e_pads[lvl])

    deg2 = [d.reshape(-1, 1) for d in degs]
    cnt2 = [c.reshape(-1, 1) for c in cnts]

    def pre_args(pre):
        return (pre['w1'], rs(pre['g1']), rs(pre['b1'])) if pre else ()

    def rb_post_stage(lvl, p, xx, h1, P, pre):
        n = _NS[lvl]
        c = xx.shape[1]

        def f(xx, h1, P, deg, wg, bg, g2, b2, w2, g3, b3, *nx):
            agg = (P[0, :n] + P[1, :n]) / (deg + 1.0)
            gg = (h1 + agg) @ wg + bg
            h2 = _norm(gg, g2, b2)
            u = _norm(h2 @ w2, g3, b3, relu=False)
            xo = jax.nn.relu(xx + u)
            if not nx:
                return xo
            w1n, g1n, b1n = nx
            return xo, _norm(xo @ w1n, g1n, b1n)

        outs = S(n, c) if not pre else [S(n, c), S(n, pre['w1'].shape[1])]
        return _tc(f, outs, xx, h1, P, deg2[lvl], p['wg'], rs(p['bg']),
                   rs(p['g2']), rs(p['b2']), p['w2'], rs(p['g3']), rs(p['b3']),
                   *pre_args(pre))

    def run_block(lvl, p, xx, h1, pre):
        P = seg_p(h1, lvl)
        return rb_post_stage(lvl, p, xx, h1, P, pre)

    pc = params['conv1']
    y = _tc(lambda xx, w: xx @ w, S(_NS[0], 32), x, pc['w'])
    P = seg_p(y, 0)
    blk0 = params['enc'][0][0]

    def conv1_post(y, P, deg, b, g, bt, w1n, g1n, b1n):
        agg = (P[0, :_NS[0]] + P[1, :_NS[0]]) / (deg + 1.0)
        h = _norm(y + agg + b, g, bt)
        return h, _norm(h @ w1n, g1n, b1n)

    cur, h1 = _tc(conv1_post, [S(_NS[0], 32), S(_NS[0], 8)],
                  y, P, deg2[0], rs(pc['b']), rs(pc['g']), rs(pc['bt']),
                  *pre_args(blk0))

    convs = {}
    for i in range(4):
        blocks = params['enc'][i]
        for bi, blk in enumerate(blocks):
            last_of_net_level = (bi == 2)
            if not last_of_net_level:
                pre = blocks[bi + 1]
            elif i < 3:
                pre = None
            else:
                pre = params['dec'][0][0]
            res = run_block(i, blk, cur, h1, pre)
            if pre:
                cur, h1 = res
            else:
                cur = res
        convs[i] = cur
        if i < 3:
            Pp = _seg_sum_p(cur, pool_srcs[i], pool_dsts[i],
                            _NS[i + 1], pool_pads[i])
            d = params['down'][i]
            nxt = params['enc'][i + 1][0]
            n1 = _NS[i + 1]

            def pool_post(P, cnt, dw, g, bt, w1n, g1n, b1n, n1=n1):
                pooled = (P[0, :n1] + P[1, :n1]) / jnp.maximum(cnt, 1.0)
                c2 = _norm(pooled @ dw, g, bt)
                return c2, _norm(c2 @ w1n, g1n, b1n)

            cur, h1 = _tc(pool_post,
                          [S(n1, d['w'].shape[1]), S(n1, nxt['w1'].shape[1])],
                          Pp, cnt2[i], d['w'], rs(d['g']), rs(d['bt']),
                          *pre_args(nxt))

    deconv = convs[3]
    outs = []
    for i in range(4):
        L = 3 - i
        blocks = params['dec'][i]
        for bi, blk in enumerate(blocks):
            if bi < 2:
                pre = blocks[bi + 1]
            else:
                pre = None
            res = run_block(L, blk, deconv, h1, pre)
            if pre:
                deconv, h1 = res
            else:
                deconv = res
        hp, hr = params['pred'][i], params['reg'][i]
        n = _NS[L]
        if i < 3:
            u = params['up'][i]

            def head_up(xx, pw1, pg, pb, pw2, pb2, rw1, rg, rb, rw2, rb2, uw):
                logit = _norm(xx @ pw1, pg, pb) @ pw2 + pb2
                signal = jnp.tanh(_norm(xx @ rw1, rg, rb) @ rw2 + rb2)
                return jnp.concatenate([logit, signal], axis=1), xx @ uw

            o_i, t = _tc(head_up, [S(n, 6), S(n, u['w'].shape[1])],
                         deconv, hp['w1'], rs(hp['g']), rs(hp['b']), hp['w2'],
                         rs(hp['b2']), hr['w1'], rs(hr['g']), rs(hr['b']),
                         hr['w2'], rs(hr['b2']), u['w'])
            outs.append(o_i)
            up = _gather_rows(t, clusters[L - 1], _NS[L - 1])
            nxt = params['dec'][i + 1][0]
            skip = convs[L - 1]
            n1 = _NS[L - 1]

            def up_post(up, sk, g, bt, w1n, g1n, b1n):
                dc = _norm(up, g, bt) + sk
                return dc, _norm(dc @ w1n, g1n, b1n)

            deconv, h1 = _tc(up_post,
                             [S(n1, skip.shape[1]), S(n1, nxt['w1'].shape[1])],
                             up, skip, rs(u['g']), rs(u['bt']), *pre_args(nxt))
        else:

            def head_only(xx, pw1, pg, pb, pw2, pb2, rw1, rg, rb, rw2, rb2):
                logit = _norm(xx @ pw1, pg, pb) @ pw2 + pb2
                signal = jnp.tanh(_norm(xx @ rw1, rg, rb) @ rw2 + rb2)
                return jnp.concatenate([logit, signal], axis=1)

            outs.append(_tc(head_only, S(n, 6),
                            deconv, hp['w1'], rs(hp['g']), rs(hp['b']),
                            hp['w2'], rs(hp['b2']), hr['w1'], rs(hr['g']),
                            rs(hr['b']), hr['w2'], rs(hr['b2'])))
    return jnp.concatenate(outs, axis=0)
